# baseline (speedup 1.0000x reference)
import functools

import jax
import jax.numpy as jnp
from jax import lax
from jax.experimental import pallas as pl
from jax.experimental.pallas import tpu as pltpu
from jax.experimental.pallas import tpu_sc as plsc

_TOL = 0.01
_MAXITER = 16

_N = 10000
_E = 320000
_CP = 48
_CE = 2048
_CPH = 80
_EPAD = 2 * _CPH * _CE
_NB = 4


def _projT_body(w_ref, nf_ref, b_ref, o_ref):
    o_ref[...] = (
        lax.dot_general(w_ref[...], nf_ref[...], (((1,), (1,)), ((), ())),
                        preferred_element_type=jnp.float32)
        + b_ref[...]
    )


def _project_T(nf, WpT, bp):
    N, D = nf.shape
    return pl.pallas_call(
        _projT_body,
        out_shape=jax.ShapeDtypeStruct((_CP, N), jnp.float32),
    )(WpT, nf, bp.reshape(_CP, 1))


@functools.partial(
    pl.kernel,
    out_type=jax.ShapeDtypeStruct((2, _CP, _N), jnp.float32),
    mesh=plsc.VectorSubcoreMesh(core_axis_name="c", subcore_axis_name="s"),
    compiler_params=pltpu.CompilerParams(
        use_tc_tiling_on_sc=False, needs_layout_passes=False),
    scratch_types=(
        [pltpu.VMEM((_N,), jnp.float32)] * 6
        + [pltpu.VMEM((2, _CE), jnp.int32)] * _NB
        + [pltpu.VMEM((_CE,), jnp.float32)] * _NB
        + [pltpu.SemaphoreType.DMA] * _NB
    ),
)
def _sc_matvec(vT_hbm, ech_hbm, adjh_hbm, zeros_hbm, av_hbm, *scr):
    vp = scr[0:3]
    ap = scr[3:6]
    ebuf = scr[6:6 + _NB]
    abuf = scr[6 + _NB:6 + 2 * _NB]
    esem = scr[6 + 2 * _NB:6 + 3 * _NB]
    cid = lax.axis_index("c")
    sid = lax.axis_index("s")
    cbase = cid * _CPH
    for k in range(3):
        pltpu.sync_copy(vT_hbm.at[3 * sid + k], vp[k])
        pltpu.sync_copy(zeros_hbm, ap[k])
    for q in range(_NB - 1):
        pltpu.async_copy(ech_hbm.at[cbase + q], ebuf[q], esem[q])
        pltpu.async_copy(adjh_hbm.at[cbase + q], abuf[q], esem[q])

    def outer_body(o, carry):
        for b in range(_NB):
            i = o * _NB + b
            pltpu.make_async_copy(ech_hbm.at[cbase], ebuf[b], esem[b]).wait()
            pltpu.make_async_copy(adjh_hbm.at[cbase], abuf[b], esem[b]).wait()

            nxt = (b + _NB - 1) % _NB

            @pl.when(i + _NB - 1 < _CPH)
            def _():
                pltpu.async_copy(ech_hbm.at[cbase + i + _NB - 1],
                                 ebuf[nxt], esem[nxt])
                pltpu.async_copy(adjh_hbm.at[cbase + i + _NB - 1],
                                 abuf[nxt], esem[nxt])

            def vec_body(j, c2):
                colv = ebuf[b][0, pl.ds(j * 16, 16)]
                rowv = ebuf[b][1, pl.ds(j * 16, 16)]
                adjv = abuf[b][pl.ds(j * 16, 16)]
                for k in range(3):
                    g = plsc.load_gather(vp[k], [colv])
                    plsc.addupdate_scatter(ap[k], [rowv], g * adjv)
                return c2

            lax.fori_loop(0, _CE // 16, vec_body, 0, unroll=8)
        return carry

    lax.fori_loop(0, _CPH // _NB, outer_body, 0)
    for k in range(3):
        pltpu.sync_copy(ap[k], av_hbm.at[cid, 3 * sid + k])


_NIDP = 1024


@functools.partial(
    pl.kernel,
    out_type=jax.ShapeDtypeStruct((_NIDP, _CP), jnp.float32),
    mesh=plsc.VectorSubcoreMesh(core_axis_name="c", subcore_axis_name="s"),
    compiler_params=pltpu.CompilerParams(
        use_tc_tiling_on_sc=False, needs_layout_passes=False),
    scratch_types=[
        pltpu.VMEM((32,), jnp.int32),
        pltpu.VMEM((32, _CP), jnp.float32),
        pltpu.SemaphoreType.DMA,
    ],
)
def _sc_ids_gather(sol_hbm, ids_hbm, out_hbm, ids_v, rows_v, sem):
    w = lax.axis_index("c") * 16 + lax.axis_index("s")
    pltpu.sync_copy(ids_hbm.at[w], ids_v)
    pltpu.async_copy(sol_hbm.at[ids_v], rows_v, sem).wait()
    pltpu.sync_copy(rows_v, out_hbm.at[pl.ds(w * 32, 32)])


def kernel(node_features, adj_values, e0, W, b, edge_index, ids):
    D, C = W.shape
    WpT = jnp.zeros((_CP, D), jnp.float32).at[:C, :].set(W.T)
    bp = jnp.zeros((_CP,), jnp.float32).at[:C].set(b)
    xT = _project_T(node_features, WpT, bp)

    pad = _EPAD - _E
    colp = jnp.pad(edge_index[1], (0, pad)).reshape(2 * _CPH, _CE)
    rowp = jnp.pad(edge_index[0], (0, pad)).reshape(2 * _CPH, _CE)
    ech = jnp.stack([colp, rowp], axis=1)
    adjh = jnp.pad(adj_values, (0, pad)).reshape(2 * _CPH, _CE)
    zeros = jnp.zeros((_N,), jnp.float32)

    epsilon = jax.nn.sigmoid(e0)
    c = 1.0 - epsilon

    def matvec(v):
        av2 = _sc_matvec(v, ech, adjh, zeros)
        return v - c * (av2[0] + av2[1])

    sol, _ = jax.scipy.sparse.linalg.cg(matvec, xT, tol=_TOL, maxiter=_MAXITER)

    ids_p = jnp.pad(ids, (0, _NIDP - ids.shape[0])).reshape(32, 32)
    outp = _sc_ids_gather(sol.T, ids_p)
    return outp[: ids.shape[0], :C]


# speedup vs baseline: 2.7230x; 2.7230x over previous
import functools

import jax
import jax.numpy as jnp
from jax import lax
from jax.experimental import pallas as pl
from jax.experimental.pallas import tpu as pltpu
from jax.experimental.pallas import tpu_sc as plsc

_TOL = 0.01
_MAXITER = 16

_N = 10000
_E = 320000
_CP = 48
_CHUNK = 128
_NBUF = 3
_NCHUNKS = 2592
_EPAD = _NCHUNKS * _CHUNK
_RPW = _N // 16
_NC = 2


def _proj_body(nf_ref, w_ref, b_ref, o_ref):
    o_ref[...] = (
        jnp.dot(nf_ref[...], w_ref[...], preferred_element_type=jnp.float32)
        + b_ref[...]
    )


def _project(nf, Wp, bp):
    N, D = nf.shape
    BN = 2000
    return pl.pallas_call(
        _proj_body,
        grid=(N // BN,),
        in_specs=[
            pl.BlockSpec((BN, D), lambda i: (i, 0)),
            pl.BlockSpec((D, _CP), lambda i: (0, 0)),
            pl.BlockSpec((1, _CP), lambda i: (0, 0)),
        ],
        out_specs=pl.BlockSpec((BN, _CP), lambda i: (i, 0)),
        out_shape=jax.ShapeDtypeStruct((N, _CP), jnp.float32),
    )(nf, Wp, bp.reshape(1, _CP))


def _make_sc_matvec(nc):
    cpw = _NCHUNKS // (nc * 16)

    @functools.partial(
        pl.kernel,
        out_type=jax.ShapeDtypeStruct((nc, _N, _CP), jnp.float32),
        mesh=plsc.VectorSubcoreMesh(
            core_axis_name="c", subcore_axis_name="s", num_cores=nc),
        compiler_params=pltpu.CompilerParams(use_tc_tiling_on_sc=False),
        scratch_types=(
            [
                pltpu.VMEM((cpw, _CHUNK), jnp.int32),
                pltpu.VMEM((cpw, _CHUNK), jnp.int32),
                pltpu.VMEM((cpw, _CHUNK), jnp.float32),
                pltpu.VMEM_SHARED((_N, _CP), jnp.float32),
                pltpu.VMEM_SHARED((_N, _CP), jnp.float32),
            ]
            + [pltpu.VMEM((_CHUNK, _CP), jnp.float32)] * (2 * _NBUF)
            + [pltpu.SemaphoreType.DMA] * (2 * _NBUF)
        ),
    )
    def sc_matvec(vpad_hbm, col_hbm, row_hbm, adj_hbm, zeros_hbm, av_hbm,
                  col_all, row_all, adj_all, av_sh, v_sh, *bufs_and_sems):
        _sc_matvec_body(cpw, col_all, row_all, adj_all, av_sh, v_sh,
                        bufs_and_sems, vpad_hbm, col_hbm, row_hbm, adj_hbm,
                        zeros_hbm, av_hbm)

    return sc_matvec


def _sc_matvec_body(cpw, col_all, row_all, adj_all, av_sh, v_sh,
                    bufs_and_sems, vpad_hbm, col_hbm, row_hbm, adj_hbm,
                    zeros_hbm, av_hbm):
    gb = bufs_and_sems[0:_NBUF]
    sb = bufs_and_sems[_NBUF:2 * _NBUF]
    gsem = bufs_and_sems[2 * _NBUF:3 * _NBUF]
    ssem = bufs_and_sems[3 * _NBUF:4 * _NBUF]
    cid = lax.axis_index("c")
    sid = lax.axis_index("s")
    base = (cid * 16 + sid) * cpw
    pltpu.sync_copy(col_hbm.at[pl.ds(base, cpw)], col_all)
    pltpu.sync_copy(row_hbm.at[pl.ds(base, cpw)], row_all)
    pltpu.sync_copy(adj_hbm.at[pl.ds(base, cpw)], adj_all)
    pltpu.sync_copy(zeros_hbm.at[pl.ds(sid * _RPW, _RPW)],
                    av_sh.at[pl.ds(sid * _RPW, _RPW)])
    pltpu.sync_copy(vpad_hbm.at[pl.ds(sid * _RPW, _RPW)],
                    v_sh.at[pl.ds(sid * _RPW, _RPW)])
    plsc.subcore_barrier()

    for b in range(_NBUF):
        pltpu.async_copy(v_sh.at[col_all.at[b]], gb[b], gsem[b])

    n_outer = cpw // _NBUF

    def outer_body(o, carry):
        for b in range(_NBUF):
            i = o * _NBUF + b
            pltpu.make_async_copy(v_sh.at[col_all.at[b]], gb[b],
                                  gsem[b]).wait()
            @pl.when(o > 0)
            def _():
                pltpu.make_async_copy(sb[b], av_sh.at[row_all.at[b]],
                                      ssem[b]).wait()

            def edge_body(e16, c2):
                a16 = adj_all[i, pl.ds(e16 * 16, 16)]
                for j in range(16):
                    e = e16 * 16 + j
                    a = a16[j]
                    for k in range(3):
                        sb[b][e, pl.ds(16 * k, 16)] = (
                            gb[b][e, pl.ds(16 * k, 16)] * a
                        )
                return c2

            lax.fori_loop(0, _CHUNK // 16, edge_body, 0)

            @pl.when(o < n_outer - 1)
            def _():
                pltpu.async_copy(v_sh.at[col_all.at[i + _NBUF]],
                                 gb[b], gsem[b])

            pltpu.async_copy(sb[b], av_sh.at[row_all.at[i]], ssem[b],
                             add=True)
        return carry

    lax.fori_loop(0, n_outer, outer_body, 0)
    for b in range(_NBUF):
        pltpu.make_async_copy(sb[b], av_sh.at[row_all.at[b]],
                              ssem[b]).wait()
    plsc.subcore_barrier()
    pltpu.sync_copy(av_sh.at[pl.ds(sid * _RPW, _RPW)],
                    av_hbm.at[cid, pl.ds(sid * _RPW, _RPW)])


_NIDP = 1024


@functools.partial(
    pl.kernel,
    out_type=jax.ShapeDtypeStruct((_NIDP, _CP), jnp.float32),
    mesh=plsc.VectorSubcoreMesh(core_axis_name="c", subcore_axis_name="s"),
    compiler_params=pltpu.CompilerParams(use_tc_tiling_on_sc=False),
    scratch_types=[
        pltpu.VMEM((32,), jnp.int32),
        pltpu.VMEM((32, _CP), jnp.float32),
        pltpu.SemaphoreType.DMA,
    ],
)
def _sc_ids_gather(sol_hbm, ids_hbm, out_hbm, ids_v, rows_v, sem):
    w = lax.axis_index("c") * 16 + lax.axis_index("s")
    pltpu.sync_copy(ids_hbm.at[w], ids_v)
    pltpu.async_copy(sol_hbm.at[ids_v], rows_v, sem).wait()
    pltpu.sync_copy(rows_v, out_hbm.at[pl.ds(w * 32, 32)])


def kernel(node_features, adj_values, e0, W, b, edge_index, ids):
    D, C = W.shape
    Wp = jnp.zeros((D, _CP), jnp.float32).at[:, :C].set(W)
    bp = jnp.zeros((_CP,), jnp.float32).at[:C].set(b)
    xpad = _project(node_features, Wp, bp)

    row = jnp.pad(edge_index[0], (0, _EPAD - _E)).reshape(_NCHUNKS, _CHUNK)
    col = jnp.pad(edge_index[1], (0, _EPAD - _E)).reshape(_NCHUNKS, _CHUNK)
    adj = jnp.pad(adj_values, (0, _EPAD - _E)).reshape(_NCHUNKS, _CHUNK)
    zeros = jnp.zeros((_N, _CP), jnp.float32)

    epsilon = jax.nn.sigmoid(e0)
    c = 1.0 - epsilon
    mv = _make_sc_matvec(_NC)

    def matvec(v):
        av2 = mv(v, col, row, adj, zeros)
        av = av2[0]
        for i in range(1, _NC):
            av = av + av2[i]
        return v - c * av

    sol, _ = jax.scipy.sparse.linalg.cg(matvec, xpad, tol=_TOL, maxiter=_MAXITER)

    ids_p = jnp.pad(ids, (0, _NIDP - ids.shape[0])).reshape(32, 32)
    outp = _sc_ids_gather(sol, ids_p)
    return outp[: ids.shape[0], :C]


# speedup vs baseline: 3.1718x; 1.1648x over previous
import functools

import jax
import jax.numpy as jnp
from jax import lax
from jax.experimental import pallas as pl
from jax.experimental.pallas import tpu as pltpu
from jax.experimental.pallas import tpu_sc as plsc

_TOL = 0.01
_MAXITER = 16

_N = 10000
_E = 320000
_CP = 48
_CHUNK = 128
_NBUF = 3
_NCHUNKS = 2592
_EPAD = _NCHUNKS * _CHUNK
_RPW = _N // 16
_NC = 2


_BN = 2000


def _proj_body(nf_ref, w_ref, b_ref, o_ref, bb_ref):
    acc = (
        jnp.dot(nf_ref[...], w_ref[...], preferred_element_type=jnp.float32)
        + b_ref[...]
    )
    o_ref[...] = acc

    @pl.when(pl.program_id(0) == 0)
    def _():
        bb_ref[0, 0] = 0.0

    bb_ref[0, 0] += jnp.sum(acc * acc)


def _project(nf, Wp, bp):
    N, D = nf.shape
    return pl.pallas_call(
        _proj_body,
        grid=(N // _BN,),
        in_specs=[
            pl.BlockSpec((_BN, D), lambda i: (i, 0)),
            pl.BlockSpec((D, _CP), lambda i: (0, 0)),
            pl.BlockSpec((1, _CP), lambda i: (0, 0)),
        ],
        out_specs=[
            pl.BlockSpec((_BN, _CP), lambda i: (i, 0)),
            pl.BlockSpec(memory_space=pltpu.SMEM),
        ],
        out_shape=[
            jax.ShapeDtypeStruct((N, _CP), jnp.float32),
            jax.ShapeDtypeStruct((1, 1), jnp.float32),
        ],
    )(nf, Wp, bp.reshape(1, _CP))


def _cg_update_body(x_ref, r_ref, p_ref, av_ref, gamma_ref, c_ref,
                    xo_ref, ro_ref, po_ref, go_ref, ap_scr, pap_scr, gn_scr):
    ph = pl.program_id(0)
    j = pl.program_id(1)

    @pl.when(ph == 0)
    def _():
        @pl.when(j == 0)
        def _():
            pap_scr[0, 0] = 0.0

        pblk = p_ref[...]
        ap = pblk - c_ref[0, 0] * (av_ref[0] + av_ref[1])
        ap_scr[pl.ds(j * _BN, _BN), :] = ap
        pap_scr[0, 0] += jnp.sum(pblk * ap)
        xo_ref[...] = x_ref[...]
        ro_ref[...] = r_ref[...]
        po_ref[...] = pblk

    @pl.when(ph == 1)
    def _():
        @pl.when(j == 0)
        def _():
            gn_scr[0, 0] = 0.0

        alpha = gamma_ref[0, 0] / pap_scr[0, 0]
        xo_ref[...] = x_ref[...] + alpha * p_ref[...]
        rnew = r_ref[...] - alpha * ap_scr[pl.ds(j * _BN, _BN), :]
        ro_ref[...] = rnew
        gn_scr[0, 0] += jnp.sum(rnew * rnew)
        po_ref[...] = p_ref[...]

    @pl.when(ph == 2)
    def _():
        beta = gn_scr[0, 0] / gamma_ref[0, 0]
        po_ref[...] = r_ref[...] + beta * p_ref[...]
        xo_ref[...] = x_ref[...]
        ro_ref[...] = r_ref[...]

        @pl.when(j == pl.num_programs(1) - 1)
        def _():
            go_ref[0, 0] = gn_scr[0, 0]


def _cg_update(x, r, p, av2, gamma, cc):
    nb = _N // _BN
    return pl.pallas_call(
        _cg_update_body,
        grid=(3, nb),
        in_specs=[
            pl.BlockSpec((_BN, _CP), lambda ph, j: (j, 0)),
            pl.BlockSpec((_BN, _CP), lambda ph, j: (j, 0)),
            pl.BlockSpec((_BN, _CP), lambda ph, j: (j, 0)),
            pl.BlockSpec((2, _BN, _CP), lambda ph, j: (0, j, 0)),
            pl.BlockSpec(memory_space=pltpu.SMEM),
            pl.BlockSpec(memory_space=pltpu.SMEM),
        ],
        out_specs=[
            pl.BlockSpec((_BN, _CP), lambda ph, j: (j, 0)),
            pl.BlockSpec((_BN, _CP), lambda ph, j: (j, 0)),
            pl.BlockSpec((_BN, _CP), lambda ph, j: (j, 0)),
            pl.BlockSpec(memory_space=pltpu.SMEM),
        ],
        out_shape=[
            jax.ShapeDtypeStruct((_N, _CP), jnp.float32),
            jax.ShapeDtypeStruct((_N, _CP), jnp.float32),
            jax.ShapeDtypeStruct((_N, _CP), jnp.float32),
            jax.ShapeDtypeStruct((1, 1), jnp.float32),
        ],
        scratch_shapes=[
            pltpu.VMEM((_N, _CP), jnp.float32),
            pltpu.SMEM((1, 1), jnp.float32),
            pltpu.SMEM((1, 1), jnp.float32),
        ],
        input_output_aliases={0: 0, 1: 1, 2: 2},
    )(x, r, p, av2, gamma, cc)


def _make_sc_matvec(nc):
    cpw = _NCHUNKS // (nc * 16)

    @functools.partial(
        pl.kernel,
        out_type=jax.ShapeDtypeStruct((nc, _N, _CP), jnp.float32),
        mesh=plsc.VectorSubcoreMesh(
            core_axis_name="c", subcore_axis_name="s", num_cores=nc),
        compiler_params=pltpu.CompilerParams(use_tc_tiling_on_sc=False),
        scratch_types=(
            [
                pltpu.VMEM((cpw, _CHUNK), jnp.int32),
                pltpu.VMEM((cpw, _CHUNK), jnp.int32),
                pltpu.VMEM((cpw, _CHUNK), jnp.float32),
                pltpu.VMEM_SHARED((_N, _CP), jnp.float32),
                pltpu.VMEM_SHARED((_N, _CP), jnp.float32),
            ]
            + [pltpu.VMEM((_CHUNK, _CP), jnp.float32)] * (2 * _NBUF)
            + [pltpu.SemaphoreType.DMA] * (2 * _NBUF)
        ),
    )
    def sc_matvec(vpad_hbm, col_hbm, row_hbm, adj_hbm, zeros_hbm, av_hbm,
                  col_all, row_all, adj_all, av_sh, v_sh, *bufs_and_sems):
        _sc_matvec_body(cpw, col_all, row_all, adj_all, av_sh, v_sh,
                        bufs_and_sems, vpad_hbm, col_hbm, row_hbm, adj_hbm,
                        zeros_hbm, av_hbm)

    return sc_matvec


def _sc_matvec_body(cpw, col_all, row_all, adj_all, av_sh, v_sh,
                    bufs_and_sems, vpad_hbm, col_hbm, row_hbm, adj_hbm,
                    zeros_hbm, av_hbm):
    gb = bufs_and_sems[0:_NBUF]
    sb = bufs_and_sems[_NBUF:2 * _NBUF]
    gsem = bufs_and_sems[2 * _NBUF:3 * _NBUF]
    ssem = bufs_and_sems[3 * _NBUF:4 * _NBUF]
    cid = lax.axis_index("c")
    sid = lax.axis_index("s")
    base = (cid * 16 + sid) * cpw
    pltpu.sync_copy(col_hbm.at[pl.ds(base, cpw)], col_all)
    pltpu.sync_copy(row_hbm.at[pl.ds(base, cpw)], row_all)
    pltpu.sync_copy(adj_hbm.at[pl.ds(base, cpw)], adj_all)
    pltpu.sync_copy(zeros_hbm.at[pl.ds(sid * _RPW, _RPW)],
                    av_sh.at[pl.ds(sid * _RPW, _RPW)])
    pltpu.sync_copy(vpad_hbm.at[pl.ds(sid * _RPW, _RPW)],
                    v_sh.at[pl.ds(sid * _RPW, _RPW)])
    plsc.subcore_barrier()

    for b in range(_NBUF):
        pltpu.async_copy(v_sh.at[col_all.at[b]], gb[b], gsem[b])

    n_outer = cpw // _NBUF

    def outer_body(o, carry):
        for b in range(_NBUF):
            i = o * _NBUF + b
            pltpu.make_async_copy(v_sh.at[col_all.at[b]], gb[b],
                                  gsem[b]).wait()
            @pl.when(o > 0)
            def _():
                pltpu.make_async_copy(sb[b], av_sh.at[row_all.at[b]],
                                      ssem[b]).wait()

            def edge_body(e16, c2):
                a16 = adj_all[i, pl.ds(e16 * 16, 16)]
                for j in range(16):
                    e = e16 * 16 + j
                    a = a16[j]
                    for k in range(3):
                        sb[b][e, pl.ds(16 * k, 16)] = (
                            gb[b][e, pl.ds(16 * k, 16)] * a
                        )
                return c2

            lax.fori_loop(0, _CHUNK // 16, edge_body, 0)

            @pl.when(o < n_outer - 1)
            def _():
                pltpu.async_copy(v_sh.at[col_all.at[i + _NBUF]],
                                 gb[b], gsem[b])

            pltpu.async_copy(sb[b], av_sh.at[row_all.at[i]], ssem[b],
                             add=True)
        return carry

    lax.fori_loop(0, n_outer, outer_body, 0)
    for b in range(_NBUF):
        pltpu.make_async_copy(sb[b], av_sh.at[row_all.at[b]],
                              ssem[b]).wait()
    plsc.subcore_barrier()
    pltpu.sync_copy(av_sh.at[pl.ds(sid * _RPW, _RPW)],
                    av_hbm.at[cid, pl.ds(sid * _RPW, _RPW)])


_NIDP = 1024


@functools.partial(
    pl.kernel,
    out_type=jax.ShapeDtypeStruct((_NIDP, _CP), jnp.float32),
    mesh=plsc.VectorSubcoreMesh(core_axis_name="c", subcore_axis_name="s"),
    compiler_params=pltpu.CompilerParams(use_tc_tiling_on_sc=False),
    scratch_types=[
        pltpu.VMEM((32,), jnp.int32),
        pltpu.VMEM((32, _CP), jnp.float32),
        pltpu.SemaphoreType.DMA,
    ],
)
def _sc_ids_gather(sol_hbm, ids_hbm, out_hbm, ids_v, rows_v, sem):
    w = lax.axis_index("c") * 16 + lax.axis_index("s")
    pltpu.sync_copy(ids_hbm.at[w], ids_v)
    pltpu.async_copy(sol_hbm.at[ids_v], rows_v, sem).wait()
    pltpu.sync_copy(rows_v, out_hbm.at[pl.ds(w * 32, 32)])


def kernel(node_features, adj_values, e0, W, b, edge_index, ids):
    D, C = W.shape
    Wp = jnp.zeros((D, _CP), jnp.float32).at[:, :C].set(W)
    bp = jnp.zeros((_CP,), jnp.float32).at[:C].set(b)
    xpad, bb = _project(node_features, Wp, bp)

    row = jnp.pad(edge_index[0], (0, _EPAD - _E)).reshape(_NCHUNKS, _CHUNK)
    col = jnp.pad(edge_index[1], (0, _EPAD - _E)).reshape(_NCHUNKS, _CHUNK)
    adj = jnp.pad(adj_values, (0, _EPAD - _E)).reshape(_NCHUNKS, _CHUNK)
    zeros = jnp.zeros((_N, _CP), jnp.float32)

    epsilon = jax.nn.sigmoid(e0)
    cc = (1.0 - epsilon).reshape(1, 1)
    mv = _make_sc_matvec(_NC)

    atol2 = (_TOL * _TOL) * bb[0, 0]

    def cond(carry):
        _, _, _, gamma, k = carry
        return (gamma[0, 0] > atol2) & (k < _MAXITER)

    def body(carry):
        x, r, p, gamma, k = carry
        av2 = mv(p, col, row, adj, zeros)
        x, r, p, gamma = _cg_update(x, r, p, av2, gamma, cc)
        return x, r, p, gamma, k + 1

    x0 = jnp.zeros((_N, _CP), jnp.float32)
    sol, _, _, _, _ = lax.while_loop(
        cond, body, (x0, xpad, xpad, bb, jnp.int32(0)))

    ids_p = jnp.pad(ids, (0, _NIDP - ids.shape[0])).reshape(32, 32)
    outp = _sc_ids_gather(sol, ids_p)
    return outp[: ids.shape[0], :C]
